# baseline (speedup 1.0000x reference)
import functools

import jax
import jax.numpy as jnp
from jax import lax
from jax.experimental import pallas as pl
from jax.experimental.pallas import tpu as pltpu
from jax.experimental.pallas import tpu_sc as plsc

_B = 16384
_D = 64
_NNEG = 5
_NC = 2
_NS = 16
_NW = _NC * _NS
_BPW = _B // _NW
_CH = 64
_NCHUNK = _BPW // _CH


def _sc_scores(pos_u, pos_v, neg_t, U, V):
    mesh = plsc.VectorSubcoreMesh(core_axis_name="c", subcore_axis_name="s")

    buf_types = (
        [pltpu.VMEM((_CH,), jnp.int32) for _ in range(7)]
        + [
            pltpu.VMEM((_CH, 128), jnp.float32),
            pltpu.VMEM((_CH, 128), jnp.float32),
            pltpu.VMEM((_NNEG * _CH, 128), jnp.float32),
            pltpu.SemaphoreType.DMA,
        ]
    )

    @functools.partial(
        pl.kernel,
        mesh=mesh,
        compiler_params=pltpu.CompilerParams(needs_layout_passes=False),
        out_type=jax.ShapeDtypeStruct((8, _B), jnp.float32),
        scratch_types=buf_types + buf_types + [
            pltpu.VMEM((8, 2 * _CH), jnp.float32),
        ],
    )
    def k(pos_u_hbm, pos_v_hbm, neg_hbm, u_hbm, v_hbm, out_hbm, *scratch):
        bufs = (scratch[0:11], scratch[11:22])
        scores = scratch[22]
        wid = lax.axis_index("s") * _NC + lax.axis_index("c")
        base = wid * _BPW
        lane = lax.iota(jnp.int32, 16)

        def stage(c, buf):
            iu, iv, in0, in1, in2, in3, in4 = buf[0:7]
            u_rows, v_rows, n_rows, sem = buf[7:11]
            idx_n = (in0, in1, in2, in3, in4)
            gb = base + c * _CH
            pltpu.sync_copy(pos_u_hbm.at[pl.ds(gb, _CH)], iu)
            pltpu.sync_copy(pos_v_hbm.at[pl.ds(gb, _CH)], iv)
            for j in range(_NNEG):
                pltpu.sync_copy(neg_hbm.at[pl.ds(j * _B + gb, _CH)],
                                idx_n[j])
            pltpu.async_copy(u_hbm.at[iu], u_rows, sem)
            pltpu.async_copy(v_hbm.at[iv], v_rows, sem)
            for j in range(_NNEG):
                pltpu.async_copy(v_hbm.at[idx_n[j]],
                                 n_rows.at[pl.ds(j * _CH, _CH), :], sem)

        def drain(buf):
            u_rows, v_rows, n_rows, sem = buf[7:11]
            pltpu.make_async_copy(u_hbm.at[pl.ds(0, _CH), :], u_rows,
                                  sem).wait()
            pltpu.make_async_copy(v_hbm.at[pl.ds(0, _CH), :], v_rows,
                                  sem).wait()
            pltpu.make_async_copy(v_hbm.at[pl.ds(0, _NNEG * _CH), :],
                                  n_rows, sem).wait()

        def compute(c, buf):
            u_rows, v_rows, n_rows = buf[7:10]
            half = (c % 2) * _CH
            zero = jnp.zeros((16,), jnp.float32)

            def group(g, carry):
                s0 = g * 16
                rvec = s0 + lane

                def dstep(d, accs):
                    cvec = jnp.full((16,), d, jnp.int32)
                    uu = plsc.load_gather(u_rows, [rvec, cvec])
                    vv = plsc.load_gather(v_rows, [rvec, cvec])
                    new = [accs[0] + uu * vv]
                    for j in range(_NNEG):
                        nn = plsc.load_gather(n_rows,
                                              [j * _CH + rvec, cvec])
                        new.append(accs[1 + j] + uu * nn)
                    return tuple(new)

                accs = lax.fori_loop(0, _D, dstep, (zero,) * 6)
                for t in range(6):
                    scores[t, pl.ds(half + s0, 16)] = accs[t]
                return carry

            lax.fori_loop(0, _CH // 16, group, 0)
            if c % 2 == 1:
                pltpu.sync_copy(
                    scores,
                    out_hbm.at[:, pl.ds(base + (c - 1) * _CH, 2 * _CH)])

        stage(0, bufs[0])
        for c in range(_NCHUNK):
            buf = bufs[c % 2]
            drain(buf)
            if c + 1 < _NCHUNK:
                stage(c + 1, bufs[(c + 1) % 2])
            compute(c, buf)

    return k(pos_u, pos_v, neg_t, U, V)


def _tc_loss(s_ref, out_ref):
    s = s_ref[...]
    pos = jnp.clip(s[0, :], -10.0, 10.0)
    pos_loss = jnp.logaddexp(0.0, -pos)
    neg = jnp.clip(s[1:1 + _NNEG, :], -10.0, 10.0)
    neg_loss = jnp.logaddexp(0.0, neg)
    total = jnp.sum(pos_loss) + jnp.sum(neg_loss)
    out_ref[...] = jnp.full((1, 1), total / _B, dtype=jnp.float32)


def kernel(pos_u, pos_v, neg_v, U, V):
    pos_u = pos_u.astype(jnp.int32)
    pos_v = pos_v.astype(jnp.int32)
    neg_t = neg_v.astype(jnp.int32).T.reshape(-1)
    U_pad = jnp.pad(U, ((0, 0), (0, 128 - _D)))
    V_pad = jnp.pad(V, ((0, 0), (0, 128 - _D)))
    scores = _sc_scores(pos_u, pos_v, neg_t, U_pad, V_pad)
    loss = pl.pallas_call(
        _tc_loss,
        out_shape=jax.ShapeDtypeStruct((1, 1), jnp.float32),
    )(scores)
    return jnp.reshape(loss, ())


# speedup vs baseline: 1.4011x; 1.4011x over previous
import functools

import jax
import jax.numpy as jnp
from jax import lax
from jax.experimental import pallas as pl
from jax.experimental.pallas import tpu as pltpu
from jax.experimental.pallas import tpu_sc as plsc

_B = 16384
_D = 64
_NNEG = 5
_NC = 2
_NS = 16
_NW = _NC * _NS
_BPW = _B // _NW
_CH = 64
_NCHUNK = _BPW // _CH


def _sc_scores(pos_u, pos_v, neg_t, U, V):
    mesh = plsc.VectorSubcoreMesh(core_axis_name="c", subcore_axis_name="s")

    buf_types = [
        pltpu.VMEM((7 * _CH,), jnp.int32),
        pltpu.VMEM((_CH, _D), jnp.float32),
        pltpu.VMEM((_CH, _D), jnp.float32),
        pltpu.VMEM((_NNEG * _CH, _D), jnp.float32),
        pltpu.SemaphoreType.DMA,
    ]

    @functools.partial(
        pl.kernel,
        mesh=mesh,
        compiler_params=pltpu.CompilerParams(needs_layout_passes=False),
        out_type=jax.ShapeDtypeStruct((8, _B), jnp.float32),
        scratch_types=buf_types + buf_types + [
            pltpu.VMEM((8, 2 * _CH), jnp.float32),
        ],
    )
    def k(pos_u_hbm, pos_v_hbm, neg_hbm, u_hbm, v_hbm, out_hbm,
          idx_a, u_a, v_a, n_a, sem_a,
          idx_b, u_b, v_b, n_b, sem_b, scores):
        bufs = ((idx_a, u_a, v_a, n_a, sem_a),
                (idx_b, u_b, v_b, n_b, sem_b))
        wid = lax.axis_index("s") * _NC + lax.axis_index("c")
        base = wid * _BPW
        lane = lax.iota(jnp.int32, 16)

        def stage(c, buf):
            idx_stage, u_rows, v_rows, n_rows, sem = buf
            gb = base + c * _CH
            pltpu.sync_copy(pos_u_hbm.at[pl.ds(gb, _CH)],
                            idx_stage.at[pl.ds(0, _CH)])
            pltpu.sync_copy(pos_v_hbm.at[pl.ds(gb, _CH)],
                            idx_stage.at[pl.ds(_CH, _CH)])
            for j in range(_NNEG):
                pltpu.sync_copy(neg_hbm.at[pl.ds(j * _B + gb, _CH)],
                                idx_stage.at[pl.ds((2 + j) * _CH, _CH)])

            def issue(g, carry):
                vu = idx_stage[pl.ds(g * 16, 16)]
                vv = idx_stage[pl.ds(_CH + g * 16, 16)]
                vn = [idx_stage[pl.ds((2 + j) * _CH + g * 16, 16)]
                      for j in range(_NNEG)]
                for l in range(16):
                    b = g * 16 + l
                    pltpu.async_copy(u_hbm.at[pl.ds(vu[l], 1), :],
                                     u_rows.at[pl.ds(b, 1), :], sem)
                    pltpu.async_copy(v_hbm.at[pl.ds(vv[l], 1), :],
                                     v_rows.at[pl.ds(b, 1), :], sem)
                    for j in range(_NNEG):
                        pltpu.async_copy(
                            v_hbm.at[pl.ds(vn[j][l], 1), :],
                            n_rows.at[pl.ds(j * _CH + b, 1), :], sem)
                return carry

            lax.fori_loop(0, _CH // 16, issue, 0)

        def drain(buf):
            _, u_rows, v_rows, n_rows, sem = buf
            pltpu.make_async_copy(u_hbm.at[pl.ds(0, _CH), :], u_rows,
                                  sem).wait()
            pltpu.make_async_copy(v_hbm.at[pl.ds(0, _CH), :], v_rows,
                                  sem).wait()
            pltpu.make_async_copy(v_hbm.at[pl.ds(0, _NNEG * _CH), :],
                                  n_rows, sem).wait()

        def compute(c, buf):
            _, u_rows, v_rows, n_rows, _ = buf
            half = (c % 2) * _CH
            zero = jnp.zeros((16,), jnp.float32)

            def group(g, carry):
                s0 = g * 16
                rvec = s0 + lane

                def dstep(d, accs):
                    cvec = jnp.full((16,), d, jnp.int32)
                    uu = plsc.load_gather(u_rows, [rvec, cvec])
                    vv = plsc.load_gather(v_rows, [rvec, cvec])
                    new = [accs[0] + uu * vv]
                    for j in range(_NNEG):
                        nn = plsc.load_gather(n_rows,
                                              [j * _CH + rvec, cvec])
                        new.append(accs[1 + j] + uu * nn)
                    return tuple(new)

                accs = lax.fori_loop(0, _D, dstep, (zero,) * 6,
                                     unroll=8)
                for t in range(6):
                    scores[t, pl.ds(half + s0, 16)] = accs[t]
                return carry

            lax.fori_loop(0, _CH // 16, group, 0)
            if c % 2 == 1:
                pltpu.sync_copy(
                    scores,
                    out_hbm.at[:, pl.ds(base + (c - 1) * _CH, 2 * _CH)])

        stage(0, bufs[0])
        for c in range(_NCHUNK):
            buf = bufs[c % 2]
            drain(buf)
            if c + 1 < _NCHUNK:
                stage(c + 1, bufs[(c + 1) % 2])
            compute(c, buf)

    return k(pos_u, pos_v, neg_t, U, V)


def _tc_loss(s_ref, out_ref):
    s = s_ref[...]
    pos = jnp.clip(s[0, :], -10.0, 10.0)
    pos_loss = jnp.logaddexp(0.0, -pos)
    neg = jnp.clip(s[1:1 + _NNEG, :], -10.0, 10.0)
    neg_loss = jnp.logaddexp(0.0, neg)
    total = jnp.sum(pos_loss) + jnp.sum(neg_loss)
    out_ref[...] = jnp.full((1, 1), total / _B, dtype=jnp.float32)


def kernel(pos_u, pos_v, neg_v, U, V):
    pos_u = pos_u.astype(jnp.int32)
    pos_v = pos_v.astype(jnp.int32)
    neg_t = neg_v.astype(jnp.int32).T.reshape(-1)
    scores = _sc_scores(pos_u, pos_v, neg_t, U, V)
    loss = pl.pallas_call(
        _tc_loss,
        out_shape=jax.ShapeDtypeStruct((1, 1), jnp.float32),
    )(scores)
    return jnp.reshape(loss, ())


# speedup vs baseline: 1.4382x; 1.0265x over previous
import functools

import jax
import jax.numpy as jnp
from jax import lax
from jax.experimental import pallas as pl
from jax.experimental.pallas import tpu as pltpu
from jax.experimental.pallas import tpu_sc as plsc

_B = 16384
_D = 64
_NNEG = 5
_NC = 2
_NS = 16
_NW = _NC * _NS
_BPW = _B // _NW
_CH = 64
_NCHUNK = _BPW // _CH


def _sc_scores(pos_u, pos_v, neg_t, U, V):
    mesh = plsc.VectorSubcoreMesh(core_axis_name="c", subcore_axis_name="s")

    buf_types = [
        pltpu.VMEM((_CH, _D), jnp.float32),
        pltpu.VMEM((_CH, _D), jnp.float32),
        pltpu.VMEM((_NNEG * _CH, _D), jnp.float32),
        pltpu.SemaphoreType.DMA,
    ]

    @functools.partial(
        pl.kernel,
        mesh=mesh,
        compiler_params=pltpu.CompilerParams(needs_layout_passes=False),
        out_type=jax.ShapeDtypeStruct((8, _B), jnp.float32),
        scratch_types=buf_types + buf_types + [
            pltpu.VMEM((7 * _BPW,), jnp.int32),
            pltpu.VMEM((8, 2 * _CH), jnp.float32),
        ],
    )
    def k(pos_u_hbm, pos_v_hbm, neg_hbm, u_hbm, v_hbm, out_hbm,
          u_a, v_a, n_a, sem_a,
          u_b, v_b, n_b, sem_b, idx_all, scores):
        bufs = ((u_a, v_a, n_a, sem_a),
                (u_b, v_b, n_b, sem_b))
        wid = lax.axis_index("s") * _NC + lax.axis_index("c")
        base = wid * _BPW
        lane = lax.iota(jnp.int32, 16)

        def stage(c, buf):
            u_rows, v_rows, n_rows, sem = buf
            c0 = c * _CH

            def issue(g, carry):
                vu = idx_all[pl.ds(c0 + g * 16, 16)]
                vv = idx_all[pl.ds(_BPW + c0 + g * 16, 16)]
                vn = [idx_all[pl.ds((2 + j) * _BPW + c0 + g * 16, 16)]
                      for j in range(_NNEG)]
                for l in range(16):
                    b = g * 16 + l
                    pltpu.async_copy(u_hbm.at[pl.ds(vu[l], 1), :],
                                     u_rows.at[pl.ds(b, 1), :], sem)
                    pltpu.async_copy(v_hbm.at[pl.ds(vv[l], 1), :],
                                     v_rows.at[pl.ds(b, 1), :], sem)
                    for j in range(_NNEG):
                        pltpu.async_copy(
                            v_hbm.at[pl.ds(vn[j][l], 1), :],
                            n_rows.at[pl.ds(j * _CH + b, 1), :], sem)
                return carry

            lax.fori_loop(0, _CH // 16, issue, 0)

        def drain(buf):
            u_rows, v_rows, n_rows, sem = buf
            pltpu.make_async_copy(u_hbm.at[pl.ds(0, _CH), :], u_rows,
                                  sem).wait()
            pltpu.make_async_copy(v_hbm.at[pl.ds(0, _CH), :], v_rows,
                                  sem).wait()
            pltpu.make_async_copy(v_hbm.at[pl.ds(0, _NNEG * _CH), :],
                                  n_rows, sem).wait()

        def compute(c, buf):
            u_rows, v_rows, n_rows, _ = buf
            half = (c % 2) * _CH
            zero = jnp.zeros((16,), jnp.float32)

            def group(g, carry):
                s0 = g * 16
                rvec = s0 + lane

                def dstep(d, accs):
                    cvec = jnp.full((16,), d, jnp.int32)
                    uu = plsc.load_gather(u_rows, [rvec, cvec])
                    vv = plsc.load_gather(v_rows, [rvec, cvec])
                    new = [accs[0] + uu * vv]
                    for j in range(_NNEG):
                        nn = plsc.load_gather(n_rows,
                                              [j * _CH + rvec, cvec])
                        new.append(accs[1 + j] + uu * nn)
                    return tuple(new)

                accs = lax.fori_loop(0, _D, dstep, (zero,) * 6,
                                     unroll=8)
                for t in range(6):
                    scores[t, pl.ds(half + s0, 16)] = accs[t]
                return carry

            lax.fori_loop(0, _CH // 16, group, 0)
            if c % 2 == 1:
                pltpu.sync_copy(
                    scores,
                    out_hbm.at[:, pl.ds(base + (c - 1) * _CH, 2 * _CH)])

        pltpu.sync_copy(pos_u_hbm.at[pl.ds(base, _BPW)],
                        idx_all.at[pl.ds(0, _BPW)])
        pltpu.sync_copy(pos_v_hbm.at[pl.ds(base, _BPW)],
                        idx_all.at[pl.ds(_BPW, _BPW)])
        for j in range(_NNEG):
            pltpu.sync_copy(neg_hbm.at[pl.ds(j * _B + base, _BPW)],
                            idx_all.at[pl.ds((2 + j) * _BPW, _BPW)])
        stage(0, bufs[0])
        for c in range(_NCHUNK):
            buf = bufs[c % 2]
            drain(buf)
            if c + 1 < _NCHUNK:
                stage(c + 1, bufs[(c + 1) % 2])
            compute(c, buf)

    return k(pos_u, pos_v, neg_t, U, V)


def _tc_loss(s_ref, out_ref):
    s = s_ref[...]
    pos = jnp.clip(s[0, :], -10.0, 10.0)
    pos_loss = jnp.logaddexp(0.0, -pos)
    neg = jnp.clip(s[1:1 + _NNEG, :], -10.0, 10.0)
    neg_loss = jnp.logaddexp(0.0, neg)
    total = jnp.sum(pos_loss) + jnp.sum(neg_loss)
    out_ref[...] = jnp.full((1, 1), total / _B, dtype=jnp.float32)


def kernel(pos_u, pos_v, neg_v, U, V):
    pos_u = pos_u.astype(jnp.int32)
    pos_v = pos_v.astype(jnp.int32)
    neg_t = neg_v.astype(jnp.int32).T.reshape(-1)
    scores = _sc_scores(pos_u, pos_v, neg_t, U, V)
    loss = pl.pallas_call(
        _tc_loss,
        out_shape=jax.ShapeDtypeStruct((1, 1), jnp.float32),
    )(scores)
    return jnp.reshape(loss, ())
